# baseline (speedup 1.0000x reference)
import functools

import jax
import jax.numpy as jnp
from jax.experimental import pallas as pl
from jax.experimental.pallas import tpu as pltpu

_K = 2
_JH = 4


def _router_body(x_ref, w_ref, idx_ref, val_ref):
    E = w_ref.shape[1]
    T = x_ref.shape[0]
    logits = jnp.dot(x_ref[...], w_ref[...], preferred_element_type=jnp.float32)
    m = jnp.max(logits, axis=-1, keepdims=True)
    ex = jnp.exp(logits - m)
    aff = ex / jnp.sum(ex, axis=-1, keepdims=True)
    lane = jax.lax.broadcasted_iota(jnp.int32, (T, E), 1)
    v1 = jnp.max(aff, axis=-1, keepdims=True)
    i1 = jnp.min(jnp.where(aff == v1, lane, E), axis=-1, keepdims=True)
    aff2 = jnp.where(lane == i1, -1.0, aff)
    v2 = jnp.max(aff2, axis=-1, keepdims=True)
    i2 = jnp.min(jnp.where(aff2 == v2, lane, E), axis=-1, keepdims=True)
    idx_ref[...] = jnp.concatenate([i1, i2], axis=-1)
    val_ref[...] = jnp.concatenate([v1, v2], axis=-1)


def _mlp_body(e_ref, v_ref, xb_ref, gu_ref, dw_ref, o_ref, acc_ref):
    p = pl.program_id(0)
    s = pl.program_id(1)
    T = o_ref.shape[0]
    I2 = acc_ref.shape[1]
    I = I2 // 2

    @pl.when((p == 0) & (s == 0))
    def _():
        o_ref[...] = jnp.zeros_like(o_ref)

    @pl.when(s == 0)
    def _():
        acc_ref[...] = jnp.zeros_like(acc_ref)

    @pl.when(s < _JH)
    def _():
        acc_ref[...] += jnp.dot(
            xb_ref[0, 0], gu_ref[0, 0], preferred_element_type=jnp.float32
        )

    @pl.when(s == _JH)
    def _():
        g = acc_ref[:, :I]
        u = acc_ref[:, I:]
        a = g * jax.lax.logistic(g) * u
        part = jnp.dot(a, dw_ref[0], preferred_element_type=jnp.float32)
        t = p // _K
        scale = v_ref[p]
        rows = jax.lax.broadcasted_iota(jnp.int32, (T, 1), 0)
        o_ref[...] += jnp.where(rows == t, scale * part, 0.0)


def kernel(hidden_states, router_weight, gate_up_weights, down_weights):
    B, S, H = hidden_states.shape
    E = router_weight.shape[1]
    I = gate_up_weights.shape[2] // 2
    T = B * S
    P = T * _K
    bH = H // _JH
    x = hidden_states.reshape(T, H).astype(jnp.float32)

    idx, vals = pl.pallas_call(
        _router_body,
        out_shape=(
            jax.ShapeDtypeStruct((T, _K), jnp.int32),
            jax.ShapeDtypeStruct((T, _K), jnp.float32),
        ),
    )(x, router_weight.astype(jnp.float32))

    e_flat = idx.reshape(P)
    v_flat = vals.reshape(P)

    gu4 = gate_up_weights.reshape(E, _JH, bH, 2 * I)

    grid_spec = pltpu.PrefetchScalarGridSpec(
        num_scalar_prefetch=2,
        grid=(P, _JH + 1),
        in_specs=[
            pl.BlockSpec(
                (1, 1, 1, bH),
                lambda p, s, e, v: (p // _K, jnp.minimum(s, _JH - 1), 0, 0),
            ),
            pl.BlockSpec(
                (1, 1, bH, 2 * I),
                lambda p, s, e, v: (e[p], jnp.minimum(s, _JH - 1), 0, 0),
            ),
            pl.BlockSpec((1, I, H), lambda p, s, e, v: (e[p], 0, 0)),
        ],
        out_specs=pl.BlockSpec((T, H), lambda p, s, e, v: (0, 0)),
        scratch_shapes=[pltpu.VMEM((1, 2 * I), jnp.float32)],
    )

    out = pl.pallas_call(
        _mlp_body,
        grid_spec=grid_spec,
        out_shape=jax.ShapeDtypeStruct((T, H), jnp.float32),
        compiler_params=pltpu.CompilerParams(
            dimension_semantics=("arbitrary", "arbitrary"),
        ),
    )(e_flat, v_flat, x.reshape(T, _JH, 1, bH), gu4, down_weights)

    return out.reshape(B, S, H)


# speedup vs baseline: 1.3040x; 1.3040x over previous
import functools

import jax
import jax.numpy as jnp
from jax.experimental import pallas as pl
from jax.experimental.pallas import tpu as pltpu

_K = 2


def _router_body(x_ref, w_ref, es_ref, ps_ref, val_ref):
    E = w_ref.shape[1]
    T = x_ref.shape[0]
    P = T * _K
    logits = jnp.dot(x_ref[...], w_ref[...], preferred_element_type=jnp.float32)
    m = jnp.max(logits, axis=-1, keepdims=True)
    ex = jnp.exp(logits - m)
    aff = ex / jnp.sum(ex, axis=-1, keepdims=True)
    lane = jax.lax.broadcasted_iota(jnp.int32, (T, E), 1)
    v1 = jnp.max(aff, axis=-1, keepdims=True)
    i1 = jnp.min(jnp.where(aff == v1, lane, E), axis=-1, keepdims=True)
    aff2 = jnp.where(lane == i1, -1.0, aff)
    v2 = jnp.max(aff2, axis=-1, keepdims=True)
    i2 = jnp.min(jnp.where(aff2 == v2, lane, E), axis=-1, keepdims=True)
    val_ref[...] = jnp.concatenate([v1, v2], axis=-1)

    row = jax.lax.broadcasted_iota(jnp.int32, (T, _K), 0)
    col = jax.lax.broadcasted_iota(jnp.int32, (T, _K), 1)
    pair_id = row * _K + col
    keys = jnp.concatenate([i1, i2], axis=-1) * P + pair_id
    big = jnp.int32(1 << 20)
    out_lane = jax.lax.broadcasted_iota(jnp.int32, (1, P), 1)
    sorted_keys = jnp.zeros((1, P), jnp.int32)
    cur = keys
    for r in range(P):
        mkey = jnp.min(cur)
        sorted_keys = jnp.where(out_lane == r, mkey, sorted_keys)
        cur = jnp.where(cur == mkey, big, cur)
    es_ref[...] = sorted_keys // P
    ps_ref[...] = sorted_keys % P


def _mlp_body(e_ref, p_ref, v_ref, x_ref, g_ref, u_ref, d_ref, o_ref):
    j = pl.program_id(0)
    p = pl.program_id(1)
    T = o_ref.shape[0]

    @pl.when((j == 0) & (p == 0))
    def _():
        o_ref[...] = jnp.zeros_like(o_ref)

    xv = x_ref[0]
    g = jnp.dot(xv, g_ref[0], preferred_element_type=jnp.float32)
    u = jnp.dot(xv, u_ref[0], preferred_element_type=jnp.float32)
    a = g * jax.lax.logistic(g) * u
    part = jnp.dot(a, d_ref[0], preferred_element_type=jnp.float32)
    porig = p_ref[p]
    t = porig // _K
    scale = v_ref[porig]
    rows = jax.lax.broadcasted_iota(jnp.int32, (T, 1), 0)
    o_ref[...] += jnp.where(rows == t, scale * part, 0.0)


def kernel(hidden_states, router_weight, gate_up_weights, down_weights):
    B, S, H = hidden_states.shape
    E = router_weight.shape[1]
    I = gate_up_weights.shape[2] // 2
    T = B * S
    P = T * _K
    x = hidden_states.reshape(T, H).astype(jnp.float32)

    es, ps, vals = pl.pallas_call(
        _router_body,
        out_shape=(
            jax.ShapeDtypeStruct((1, P), jnp.int32),
            jax.ShapeDtypeStruct((1, P), jnp.int32),
            jax.ShapeDtypeStruct((T, _K), jnp.float32),
        ),
    )(x, router_weight.astype(jnp.float32))

    bI = 512
    J = I // bI

    grid_spec = pltpu.PrefetchScalarGridSpec(
        num_scalar_prefetch=3,
        grid=(J, P),
        in_specs=[
            pl.BlockSpec((1, 1, H), lambda j, p, e, q, v: (q[p] // _K, 0, 0)),
            pl.BlockSpec((1, H, bI), lambda j, p, e, q, v: (e[p], 0, j)),
            pl.BlockSpec((1, H, bI), lambda j, p, e, q, v: (e[p], 0, J + j)),
            pl.BlockSpec((1, bI, H), lambda j, p, e, q, v: (e[p], j, 0)),
        ],
        out_specs=pl.BlockSpec((T, H), lambda j, p, e, q, v: (0, 0)),
    )

    out = pl.pallas_call(
        _mlp_body,
        grid_spec=grid_spec,
        out_shape=jax.ShapeDtypeStruct((T, H), jnp.float32),
        compiler_params=pltpu.CompilerParams(
            dimension_semantics=("arbitrary", "arbitrary"),
        ),
    )(es.reshape(P), ps.reshape(P), vals.reshape(P), x.reshape(T, 1, H),
      gate_up_weights, gate_up_weights, down_weights)

    return out.reshape(B, S, H)
